# baseline (speedup 1.0000x reference)
import functools

import jax
import jax.numpy as jnp
from jax.experimental import pallas as pl
from jax.experimental.pallas import tpu as pltpu

B, S, D, H, E, K = 2, 2048, 1024, 2048, 8, 2
Z_COEF = 0.001
N = B * S
EE = E + 1

BM = 1024
BH = 512
NB_M = N // BM
NB_H = H // BH


def _gelu(x):
    return 0.5 * x * (1.0 + jax.lax.erf(x * 0.7071067811865476))


def _moe_body(x_ref, w1_ref, b1_ref, w2_ref, b2_ref, w3_ref, b3_ref,
              wts_ref, out_ref):
    e = pl.program_id(0)
    h = pl.program_id(1)
    m = pl.program_id(2)

    xb = x_ref[pl.ds(m * BM, BM), :]
    a = _gelu(jnp.dot(xb, w1_ref[0].T, preferred_element_type=jnp.float32)
              + b1_ref[0, 0][None, :])
    g = _gelu(jnp.dot(xb, w3_ref[0].T, preferred_element_type=jnp.float32)
              + b3_ref[0, 0][None, :])
    hid = (a * g).astype(jnp.bfloat16)
    contrib = jnp.dot(hid, w2_ref[0].T, preferred_element_type=jnp.float32)

    onehot = (jax.lax.broadcasted_iota(jnp.int32, (1, EE), 1) == e)
    wcol = jnp.sum(wts_ref[pl.ds(m * BM, BM), :] * onehot.astype(jnp.float32),
                   axis=1, keepdims=True)

    delta = contrib * wcol

    @pl.when(h == 0)
    def _addb2():
        out_ref[pl.ds(m * BM, BM), :] = jnp.where(
            e == 0, 0.0, out_ref[pl.ds(m * BM, BM), :]) + wcol * b2_ref[0, 0][None, :]

    out_ref[pl.ds(m * BM, BM), :] += delta


@functools.partial(jax.jit, static_argnames=())
def _moe_dense(xf, W1c, b1c, W2c, b2c, W3c, b3c, wts):
    return pl.pallas_call(
        _moe_body,
        grid=(EE, NB_H, NB_M),
        in_specs=[
            pl.BlockSpec((N, D), lambda e, h, m: (0, 0)),
            pl.BlockSpec((1, BH, D), lambda e, h, m: (e, h, 0)),
            pl.BlockSpec((1, 1, BH), lambda e, h, m: (e, 0, h)),
            pl.BlockSpec((1, D, BH), lambda e, h, m: (e, 0, h)),
            pl.BlockSpec((1, 1, D), lambda e, h, m: (e, 0, 0)),
            pl.BlockSpec((1, BH, D), lambda e, h, m: (e, h, 0)),
            pl.BlockSpec((1, 1, BH), lambda e, h, m: (e, 0, h)),
            pl.BlockSpec((N, EE), lambda e, h, m: (0, 0)),
        ],
        out_specs=pl.BlockSpec((N, D), lambda e, h, m: (0, 0)),
        out_shape=jax.ShapeDtypeStruct((N, D), jnp.float32),
        compiler_params=pltpu.CompilerParams(
            dimension_semantics=("arbitrary", "arbitrary", "arbitrary"),
        ),
    )(xf, W1c, b1c, W2c, b2c, W3c, b3c, wts)


def kernel(x, Wr, br, W1, b1, W2, b2, W3, b3, SW1, sb1, SW2, sb2, SW3, sb3):
    xf = x.reshape(N, D)

    logits = xf @ Wr.T + br
    probs = jax.nn.softmax(logits, axis=-1)
    topv, topi = jax.lax.top_k(probs, K)
    rw = topv / jnp.sum(topv, axis=-1, keepdims=True)
    oh = jax.nn.one_hot(topi, E, dtype=x.dtype)
    wts = jnp.sum(rw[..., None] * oh, axis=1)
    wts = jnp.concatenate([wts, jnp.ones((N, 1), jnp.float32)], axis=1)

    usage = jnp.mean(probs, axis=0)
    aux_loss = jnp.sum(usage * usage) * E * Z_COEF

    W1c = jnp.concatenate([W1, SW1[None]], axis=0).astype(jnp.bfloat16)
    b1c = jnp.concatenate([b1, sb1[None]], axis=0)[:, None, :]
    W2c = jnp.concatenate([W2, SW2[None]], axis=0).astype(jnp.bfloat16)
    b2c = jnp.concatenate([b2, sb2[None]], axis=0)[:, None, :]
    W3c = jnp.concatenate([W3, SW3[None]], axis=0).astype(jnp.bfloat16)
    b3c = jnp.concatenate([b3, sb3[None]], axis=0)[:, None, :]

    out = _moe_dense(xf.astype(jnp.bfloat16), W1c, b1c, W2c, b2c, W3c, b3c, wts)
    return (out.reshape(B, S, D), aux_loss)


# speedup vs baseline: 1.0556x; 1.0556x over previous
import functools

import jax
import jax.numpy as jnp
from jax.experimental import pallas as pl
from jax.experimental.pallas import tpu as pltpu

B, S, D, H, E, K = 2, 2048, 1024, 2048, 8, 2
Z_COEF = 0.001
N = B * S
EE = E + 1

BM = 1024
BH = 512
NB_M = N // BM
NB_H = H // BH


def _gelu(x):
    return 0.5 * x * (1.0 + jax.lax.erf(x * 0.7071067811865476))


def _moe_body(x_ref, w1_ref, b1_ref, w2_ref, b2_ref, w3_ref, b3_ref,
              wts_ref, out_ref):
    e = pl.program_id(0)
    h = pl.program_id(1)
    m = pl.program_id(2)

    xb = x_ref[pl.ds(m * BM, BM), :]
    a = _gelu(jnp.dot(xb, w1_ref[0].T, preferred_element_type=jnp.float32)
              + b1_ref[0, 0][None, :])
    g = _gelu(jnp.dot(xb, w3_ref[0].T, preferred_element_type=jnp.float32)
              + b3_ref[0, 0][None, :])
    contrib = jnp.dot(a * g, w2_ref[0].T, preferred_element_type=jnp.float32)

    onehot = (jax.lax.broadcasted_iota(jnp.int32, (1, EE), 1) == e)
    wcol = jnp.sum(wts_ref[pl.ds(m * BM, BM), :] * onehot.astype(jnp.float32),
                   axis=1, keepdims=True)

    delta = contrib * wcol

    @pl.when(h == 0)
    def _addb2():
        out_ref[pl.ds(m * BM, BM), :] = jnp.where(
            e == 0, 0.0, out_ref[pl.ds(m * BM, BM), :]) + wcol * b2_ref[0, 0][None, :]

    out_ref[pl.ds(m * BM, BM), :] += delta


@functools.partial(jax.jit, static_argnames=())
def _moe_dense(xf, W1c, b1c, W2c, b2c, W3c, b3c, wts):
    return pl.pallas_call(
        _moe_body,
        grid=(EE, NB_H, NB_M),
        in_specs=[
            pl.BlockSpec((N, D), lambda e, h, m: (0, 0)),
            pl.BlockSpec((1, BH, D), lambda e, h, m: (e, h, 0)),
            pl.BlockSpec((1, 1, BH), lambda e, h, m: (e, 0, h)),
            pl.BlockSpec((1, D, BH), lambda e, h, m: (e, 0, h)),
            pl.BlockSpec((1, 1, D), lambda e, h, m: (e, 0, 0)),
            pl.BlockSpec((1, BH, D), lambda e, h, m: (e, h, 0)),
            pl.BlockSpec((1, 1, BH), lambda e, h, m: (e, 0, h)),
            pl.BlockSpec((N, EE), lambda e, h, m: (0, 0)),
        ],
        out_specs=pl.BlockSpec((N, D), lambda e, h, m: (0, 0)),
        out_shape=jax.ShapeDtypeStruct((N, D), jnp.float32),
        compiler_params=pltpu.CompilerParams(
            dimension_semantics=("arbitrary", "arbitrary", "arbitrary"),
        ),
    )(xf, W1c, b1c, W2c, b2c, W3c, b3c, wts)


def kernel(x, Wr, br, W1, b1, W2, b2, W3, b3, SW1, sb1, SW2, sb2, SW3, sb3):
    xf = x.reshape(N, D)

    logits = xf @ Wr.T + br
    probs = jax.nn.softmax(logits, axis=-1)
    topv, topi = jax.lax.top_k(probs, K)
    rw = topv / jnp.sum(topv, axis=-1, keepdims=True)
    oh = jax.nn.one_hot(topi, E, dtype=x.dtype)
    wts = jnp.sum(rw[..., None] * oh, axis=1)
    wts = jnp.concatenate([wts, jnp.ones((N, 1), jnp.float32)], axis=1)

    usage = jnp.mean(probs, axis=0)
    aux_loss = jnp.sum(usage * usage) * E * Z_COEF

    W1c = jnp.concatenate([W1, SW1[None]], axis=0)
    b1c = jnp.concatenate([b1, sb1[None]], axis=0)[:, None, :]
    W2c = jnp.concatenate([W2, SW2[None]], axis=0)
    b2c = jnp.concatenate([b2, sb2[None]], axis=0)[:, None, :]
    W3c = jnp.concatenate([W3, SW3[None]], axis=0)
    b3c = jnp.concatenate([b3, sb3[None]], axis=0)[:, None, :]

    out = _moe_dense(xf, W1c, b1c, W2c, b2c, W3c, b3c, wts)
    return (out.reshape(B, S, D), aux_loss)


# speedup vs baseline: 1.5587x; 1.4766x over previous
import functools

import jax
import jax.numpy as jnp
from jax.experimental import pallas as pl
from jax.experimental.pallas import tpu as pltpu

B, S, D, H, E, K = 2, 2048, 1024, 2048, 8, 2
Z_COEF = 0.001
N = B * S
P = N * K

BM = 256
R_PAD = P + E * BM
G = R_PAD // BM

BMS = 1024
NB_S = N // BMS


def _gelu(x):
    return 0.5 * x * (1.0 + jax.lax.erf(x * 0.7071067811865476))


def _routed_body(be_ref, xs_ref, w1_ref, b1_ref, w2_ref, b2_ref,
                 w3_ref, b3_ref, w8_ref, out_ref):
    xb = xs_ref[...]
    a = _gelu(jnp.dot(xb, w1_ref[0].T, preferred_element_type=jnp.float32)
              + b1_ref[0, 0][None, :])
    g = _gelu(jnp.dot(xb, w3_ref[0].T, preferred_element_type=jnp.float32)
              + b3_ref[0, 0][None, :])
    contrib = jnp.dot(a * g, w2_ref[0].T, preferred_element_type=jnp.float32)
    wcol = w8_ref[...][:, :1]
    out_ref[...] = (contrib + b2_ref[0, 0][None, :]) * wcol


def _routed_gemm(xs, W1, b1, W2, b2, W3, b3, w8, blk_e):
    grid_spec = pltpu.PrefetchScalarGridSpec(
        num_scalar_prefetch=1,
        grid=(G,),
        in_specs=[
            pl.BlockSpec((BM, D), lambda g, be: (g, 0)),
            pl.BlockSpec((1, H, D), lambda g, be: (be[g], 0, 0)),
            pl.BlockSpec((1, 1, H), lambda g, be: (be[g], 0, 0)),
            pl.BlockSpec((1, D, H), lambda g, be: (be[g], 0, 0)),
            pl.BlockSpec((1, 1, D), lambda g, be: (be[g], 0, 0)),
            pl.BlockSpec((1, H, D), lambda g, be: (be[g], 0, 0)),
            pl.BlockSpec((1, 1, H), lambda g, be: (be[g], 0, 0)),
            pl.BlockSpec((BM, 8), lambda g, be: (g, 0)),
        ],
        out_specs=pl.BlockSpec((BM, D), lambda g, be: (g, 0)),
    )
    return pl.pallas_call(
        _routed_body,
        grid_spec=grid_spec,
        out_shape=jax.ShapeDtypeStruct((R_PAD, D), jnp.float32),
        compiler_params=pltpu.CompilerParams(
            dimension_semantics=("arbitrary",),
        ),
    )(blk_e, xs, W1, b1[:, None, :], W2, b2[:, None, :], W3, b3[:, None, :],
      w8)


def _shared_body(x_ref, w1_ref, b1_ref, w2_ref, b2_ref, w3_ref, b3_ref,
                 out_ref):
    xb = x_ref[...]
    a = _gelu(jnp.dot(xb, w1_ref[...].T, preferred_element_type=jnp.float32)
              + b1_ref[0][None, :])
    g = _gelu(jnp.dot(xb, w3_ref[...].T, preferred_element_type=jnp.float32)
              + b3_ref[0][None, :])
    out_ref[...] = (jnp.dot(a * g, w2_ref[...].T,
                            preferred_element_type=jnp.float32)
                    + b2_ref[0][None, :])


def _shared_ffn(xf, SW1, sb1, SW2, sb2, SW3, sb3):
    return pl.pallas_call(
        _shared_body,
        grid=(NB_S,),
        in_specs=[
            pl.BlockSpec((BMS, D), lambda m: (m, 0)),
            pl.BlockSpec((H, D), lambda m: (0, 0)),
            pl.BlockSpec((1, H), lambda m: (0, 0)),
            pl.BlockSpec((D, H), lambda m: (0, 0)),
            pl.BlockSpec((1, D), lambda m: (0, 0)),
            pl.BlockSpec((H, D), lambda m: (0, 0)),
            pl.BlockSpec((1, H), lambda m: (0, 0)),
        ],
        out_specs=pl.BlockSpec((BMS, D), lambda m: (m, 0)),
        out_shape=jax.ShapeDtypeStruct((N, D), jnp.float32),
        compiler_params=pltpu.CompilerParams(
            dimension_semantics=("arbitrary",),
        ),
    )(xf, SW1, sb1[None, :], SW2, sb2[None, :], SW3, sb3[None, :])


def kernel(x, Wr, br, W1, b1, W2, b2, W3, b3, SW1, sb1, SW2, sb2, SW3, sb3):
    xf = x.reshape(N, D)

    logits = xf @ Wr.T + br
    probs = jax.nn.softmax(logits, axis=-1)
    topv, topi = jax.lax.top_k(probs, K)
    rw = topv / jnp.sum(topv, axis=-1, keepdims=True)

    usage = jnp.mean(probs, axis=0)
    aux_loss = jnp.sum(usage * usage) * E * Z_COEF

    eid = topi.reshape(-1).astype(jnp.int32)
    pw = rw.reshape(-1)
    onehot = (eid[:, None] == jnp.arange(E, dtype=jnp.int32)[None, :])
    cum = jnp.cumsum(onehot.astype(jnp.int32), axis=0)
    rank = jnp.take_along_axis(cum, eid[:, None], axis=1)[:, 0] - 1
    counts = cum[-1]
    padded = ((counts + BM - 1) // BM) * BM
    offs = jnp.concatenate([jnp.zeros((1,), jnp.int32),
                            jnp.cumsum(padded)[:-1].astype(jnp.int32)])
    dest = offs[eid] + rank
    tok = (jnp.arange(P, dtype=jnp.int32) // K)

    tok_idx = jnp.zeros((R_PAD,), jnp.int32).at[dest].set(tok)
    w_pad = jnp.zeros((R_PAD,), jnp.float32).at[dest].set(pw)

    nblk = (padded // BM).astype(jnp.int32)
    blk_e = jnp.repeat(jnp.arange(E, dtype=jnp.int32), nblk,
                       total_repeat_length=G)

    xs = jnp.take(xf, tok_idx, axis=0)
    w8 = jnp.broadcast_to(w_pad[:, None], (R_PAD, 8))

    rows = _routed_gemm(xs, W1, b1, W2, b2, W3, b3, w8, blk_e)
    shared = _shared_ffn(xf, SW1, sb1, SW2, sb2, SW3, sb3)

    pos = dest.reshape(N, K)
    out = rows[pos[:, 0]] + rows[pos[:, 1]] + shared

    return (out.reshape(B, S, D), aux_loss)


# speedup vs baseline: 1.6024x; 1.0281x over previous
import functools

import jax
import jax.numpy as jnp
from jax.experimental import pallas as pl
from jax.experimental.pallas import tpu as pltpu

B, S, D, H, E, K = 2, 2048, 1024, 2048, 8, 2
Z_COEF = 0.001
N = B * S
P = N * K

BM = 256
R_PAD = P + E * BM
G = R_PAD // BM

BMS = 1024
NB_S = N // BMS


def _gelu(x):
    return 0.5 * x * (1.0 + jax.lax.erf(x * 0.7071067811865476))


def _routed_body(be_ref, xs_ref, w1_ref, b1_ref, w2_ref, b2_ref,
                 w3_ref, b3_ref, w8_ref, out_ref):
    xb = xs_ref[...]
    a = _gelu(jnp.dot(xb, w1_ref[0].T, preferred_element_type=jnp.float32)
              + b1_ref[0, 0][None, :])
    g = _gelu(jnp.dot(xb, w3_ref[0].T, preferred_element_type=jnp.float32)
              + b3_ref[0, 0][None, :])
    contrib = jnp.dot(a * g, w2_ref[0].T, preferred_element_type=jnp.float32)
    wcol = w8_ref[...][:, :1]
    out_ref[...] = (contrib + b2_ref[0, 0][None, :]) * wcol


def _routed_gemm(xs, W1, b1, W2, b2, W3, b3, w8, blk_e):
    grid_spec = pltpu.PrefetchScalarGridSpec(
        num_scalar_prefetch=1,
        grid=(G,),
        in_specs=[
            pl.BlockSpec((BM, D), lambda g, be: (g, 0)),
            pl.BlockSpec((1, H, D), lambda g, be: (be[g], 0, 0)),
            pl.BlockSpec((1, 1, H), lambda g, be: (be[g], 0, 0)),
            pl.BlockSpec((1, D, H), lambda g, be: (be[g], 0, 0)),
            pl.BlockSpec((1, 1, D), lambda g, be: (be[g], 0, 0)),
            pl.BlockSpec((1, H, D), lambda g, be: (be[g], 0, 0)),
            pl.BlockSpec((1, 1, H), lambda g, be: (be[g], 0, 0)),
            pl.BlockSpec((BM, 8), lambda g, be: (g, 0)),
        ],
        out_specs=pl.BlockSpec((BM, D), lambda g, be: (g, 0)),
    )
    return pl.pallas_call(
        _routed_body,
        grid_spec=grid_spec,
        out_shape=jax.ShapeDtypeStruct((R_PAD, D), jnp.float32),
        compiler_params=pltpu.CompilerParams(
            dimension_semantics=("arbitrary",),
        ),
    )(blk_e, xs, W1, b1[:, None, :], W2, b2[:, None, :], W3, b3[:, None, :],
      w8)


def _shared_body(x_ref, w1_ref, b1_ref, w2_ref, b2_ref, w3_ref, b3_ref,
                 out_ref):
    xb = x_ref[...]
    a = _gelu(jnp.dot(xb, w1_ref[...].T, preferred_element_type=jnp.float32)
              + b1_ref[0][None, :])
    g = _gelu(jnp.dot(xb, w3_ref[...].T, preferred_element_type=jnp.float32)
              + b3_ref[0][None, :])
    out_ref[...] = (jnp.dot(a * g, w2_ref[...].T,
                            preferred_element_type=jnp.float32)
                    + b2_ref[0][None, :])


def _shared_ffn(xf, SW1, sb1, SW2, sb2, SW3, sb3):
    return pl.pallas_call(
        _shared_body,
        grid=(NB_S,),
        in_specs=[
            pl.BlockSpec((BMS, D), lambda m: (m, 0)),
            pl.BlockSpec((H, D), lambda m: (0, 0)),
            pl.BlockSpec((1, H), lambda m: (0, 0)),
            pl.BlockSpec((D, H), lambda m: (0, 0)),
            pl.BlockSpec((1, D), lambda m: (0, 0)),
            pl.BlockSpec((H, D), lambda m: (0, 0)),
            pl.BlockSpec((1, H), lambda m: (0, 0)),
        ],
        out_specs=pl.BlockSpec((BMS, D), lambda m: (m, 0)),
        out_shape=jax.ShapeDtypeStruct((N, D), jnp.float32),
        compiler_params=pltpu.CompilerParams(
            dimension_semantics=("arbitrary",),
        ),
    )(xf, SW1, sb1[None, :], SW2, sb2[None, :], SW3, sb3[None, :])


def kernel(x, Wr, br, W1, b1, W2, b2, W3, b3, SW1, sb1, SW2, sb2, SW3, sb3):
    xf = x.reshape(N, D)

    logits = xf @ Wr.T + br
    probs = jax.nn.softmax(logits, axis=-1)
    ie = jnp.arange(E, dtype=jnp.int32)
    i1 = jnp.argmax(probs, axis=-1).astype(jnp.int32)
    v1 = jnp.max(probs, axis=-1)
    masked = jnp.where(i1[:, None] == ie[None, :], -jnp.inf, probs)
    i2 = jnp.argmax(masked, axis=-1).astype(jnp.int32)
    v2 = jnp.max(masked, axis=-1)
    denom = v1 + v2
    w1v = v1 / denom
    w2v = v2 / denom

    usage = jnp.mean(probs, axis=0)
    aux_loss = jnp.sum(usage * usage) * E * Z_COEF

    eid = jnp.stack([i1, i2], axis=1).reshape(-1)
    pw = jnp.stack([w1v, w2v], axis=1).reshape(-1)
    onehot = (eid[:, None] == ie[None, :])
    cum = jnp.cumsum(onehot.astype(jnp.int32), axis=0)
    rank = jnp.sum(jnp.where(onehot, cum, 0), axis=1) - 1
    counts = cum[-1]
    padded = ((counts + BM - 1) // BM) * BM
    offs = jnp.concatenate([jnp.zeros((1,), jnp.int32),
                            jnp.cumsum(padded)[:-1].astype(jnp.int32)])
    dest = jnp.sum(jnp.where(onehot, offs[None, :], 0), axis=1) + rank
    tok = (jnp.arange(P, dtype=jnp.int32) // K)

    tok_idx = jnp.zeros((R_PAD,), jnp.int32).at[dest].set(tok)
    w_pad = jnp.zeros((R_PAD,), jnp.float32).at[dest].set(pw)

    nblk = (padded // BM).astype(jnp.int32)
    blk_e = jnp.repeat(jnp.arange(E, dtype=jnp.int32), nblk,
                       total_repeat_length=G)

    xs = jnp.take(xf, tok_idx, axis=0)
    w8 = jnp.broadcast_to(w_pad[:, None], (R_PAD, 8))

    rows = _routed_gemm(xs, W1, b1, W2, b2, W3, b3, w8, blk_e)
    shared = _shared_ffn(xf, SW1, sb1, SW2, sb2, SW3, sb3)

    pos = dest.reshape(N, K)
    out = rows[pos[:, 0]] + rows[pos[:, 1]] + shared

    return (out.reshape(B, S, D), aux_loss)
